# Initial kernel scaffold; baseline (speedup 1.0000x reference)
import jax
import jax.numpy as jnp
from jax.experimental import pallas as pl


def kernel(input_image, V, x_data, y_data):
    raise NotImplementedError("write your pallas kernel here")


# speedup vs baseline: 7.3081x; 7.3081x over previous
import functools

import jax
import jax.numpy as jnp
from jax.experimental import pallas as pl

EPS = 1e-8


def _interp_body(x_ref, y_ref, xi_ref, v_ref, out_ref):
    B = xi_ref.shape[0]
    n = x_ref.shape[1]
    resp = [[None] * 3 for _ in range(B)]
    for c in range(3):
        xp = x_ref[c, 0]
        y0 = y_ref[c, 0]
        yp = y0
        xis = [xi_ref[b, c] for b in range(B)]
        accs = [y0 for _ in range(B)]
        for k in range(1, n):
            xk = x_ref[c, k]
            yk = y_ref[c, k]
            dxe = (xk - xp) + EPS
            dyr = (yk - yp) / dxe
            for b in range(B):
                t = xis[b] - xp
                if k == 1:
                    u = jnp.minimum(t, dxe)
                elif k == n - 1:
                    u = jnp.maximum(t, 0.0)
                else:
                    u = jnp.clip(t, 0.0, dxe)
                accs[b] = accs[b] + u * dyr
            xp = xk
            yp = yk
        for b in range(B):
            resp[b][c] = accs[b]
    for b in range(B):
        for d in range(3):
            o = (resp[b][0] * v_ref[0, d]
                 + resp[b][1] * v_ref[1, d]
                 + resp[b][2] * v_ref[2, d])
            out_ref[b, d] = jnp.clip(o, 0.0, 1.0)


@functools.partial(jax.jit, static_argnames=())
def kernel(input_image, V, x_data, y_data):
    B = input_image.shape[0]
    _, H, W, n = x_data.shape
    HW = H * W
    R = min(2048, HW)
    grid = HW // R

    x_t = jnp.transpose(x_data.reshape(3, HW, n), (0, 2, 1))
    y_t = jnp.transpose(y_data.reshape(3, HW, n), (0, 2, 1))
    v_t = jnp.transpose(V.reshape(HW, 3, 3), (1, 2, 0))
    xi = input_image.reshape(B, 3, HW)

    out = pl.pallas_call(
        _interp_body,
        grid=(grid,),
        in_specs=[
            pl.BlockSpec((3, n, R), lambda i: (0, 0, i)),
            pl.BlockSpec((3, n, R), lambda i: (0, 0, i)),
            pl.BlockSpec((B, 3, R), lambda i: (0, 0, i)),
            pl.BlockSpec((3, 3, R), lambda i: (0, 0, i)),
        ],
        out_specs=pl.BlockSpec((B, 3, R), lambda i: (0, 0, i)),
        out_shape=jax.ShapeDtypeStruct((B, 3, HW), jnp.float32),
    )(x_t, y_t, xi, v_t)
    return out.reshape(B, 3, H, W)
